# baseline (speedup 1.0000x reference)
import functools

import jax
import jax.numpy as jnp
from jax import lax
from jax.experimental import pallas as pl
from jax.experimental.pallas import tpu as pltpu
from jax.experimental.pallas import tpu_sc as plsc

N = 10000
E = 320000
G = 64
NMID = 13
NC, NS = 2, 16
NW = NC * NS
K = 80
EPW = E // NW
CPW = -(-EPW // K)
EPAD = CPW * K - EPW
N_ACC = 10112
RZ = N_ACC // NS
DA = 128
DB = 32

_f32 = jnp.float32


@functools.lru_cache(maxsize=None)
def _make_agg(first):
    mesh = plsc.VectorSubcoreMesh(core_axis_name="c", subcore_axis_name="s")

    scratch = [
        pltpu.VMEM((2, K), jnp.int32),
        pltpu.VMEM((2, K), jnp.int32),
        pltpu.VMEM((K, DA), _f32),
        pltpu.VMEM((K, DB), _f32),
        pltpu.VMEM((K, DA), _f32),
        pltpu.VMEM((K, DB), _f32),
        pltpu.VMEM_SHARED((N_ACC, DA), _f32),
        pltpu.VMEM_SHARED((N_ACC, DB), _f32),
        pltpu.SemaphoreType.DMA,
        pltpu.SemaphoreType.DMA,
        pltpu.SemaphoreType.DMA,
        pltpu.SemaphoreType.DMA,
        pltpu.SemaphoreType.DMA,
    ]

    @functools.partial(
        pl.kernel,
        out_type=[jax.ShapeDtypeStruct((NC * N_ACC, DA), _f32),
                  jax.ShapeDtypeStruct((NC * N_ACC, DA), _f32)],
        mesh=mesh,
        compiler_params=pltpu.CompilerParams(use_tc_tiling_on_sc=False),
        scratch_types=scratch,
    )
    def agg(ta_hbm, tb_hbm, sd_hbm, za_hbm, zb_hbm,
            oa_hbm, ob_hbm, id0, id1, bufa, bufa2, bufb,
            bufb2, acca, accb, sema, semb, sems, si0, si1):
        cid = lax.axis_index("c")
        sid = lax.axis_index("s")
        wid = sid * NC + cid

        def fetch(c, v, sem):
            pltpu.async_copy(sd_hbm.at[wid, c], v, sem)

        def wait_fetch(c, v, sem):
            pltpu.make_async_copy(sd_hbm.at[wid, c], v, sem).wait()

        def gather(iv, buf, buf2, sem):
            pltpu.async_copy(ta_hbm.at[iv.at[0]], buf, sem)
            if not first:
                pltpu.async_copy(tb_hbm.at[iv.at[0]], buf2, sem)

        def wait_gather(iv, buf, buf2, sem):
            pltpu.make_async_copy(ta_hbm.at[iv.at[0]], buf, sem).wait()
            if not first:
                pltpu.make_async_copy(tb_hbm.at[iv.at[0]], buf2, sem).wait()

        def scatter(buf, buf2, iv):
            pltpu.async_copy(buf, acca.at[iv.at[1]], sems, add=True)
            pltpu.async_copy(buf2, accb.at[iv.at[1]], sems, add=True)
            pltpu.make_async_copy(buf, acca.at[iv.at[1]], sems).wait()
            pltpu.make_async_copy(buf2, accb.at[iv.at[1]], sems).wait()

        if first:
            @pl.loop(0, K)
            def _(r):
                ones = jnp.full((16,), 1.0, _f32)
                bufa2[r, pl.ds(0, 16)] = ones
                bufa2[r, pl.ds(16, 16)] = ones
                bufb2[r, pl.ds(0, 16)] = ones
                bufb2[r, pl.ds(16, 16)] = ones

        fetch(0, id0, si0)
        fetch(1, id1, si1)
        pltpu.sync_copy(za_hbm.at[pl.ds(sid * RZ, RZ)],
                        acca.at[pl.ds(sid * RZ, RZ)])
        pltpu.sync_copy(zb_hbm.at[pl.ds(sid * RZ, RZ)],
                        accb.at[pl.ds(sid * RZ, RZ)])
        wait_fetch(0, id0, si0)
        gather(id0, bufa, bufa2, sema)
        plsc.subcore_barrier()

        @pl.loop(0, CPW - 1, step=2)
        def _(j):
            wait_fetch(j + 1, id1, si1)
            gather(id1, bufb, bufb2, semb)
            wait_gather(id0, bufa, bufa2, sema)
            scatter(bufa, bufa2, id0)

            @pl.when(j + 2 < CPW)
            def _():
                fetch(j + 2, id0, si0)

            wait_gather(id1, bufb, bufb2, semb)

            @pl.when(j + 2 < CPW)
            def _():
                wait_fetch(j + 2, id0, si0)
                gather(id0, bufa, bufa2, sema)

            scatter(bufb, bufb2, id1)

            @pl.when(j + 3 < CPW)
            def _():
                fetch(j + 3, id1, si1)

        wait_gather(id0, bufa, bufa2, sema)
        scatter(bufa, bufa2, id0)

        plsc.subcore_barrier()
        pltpu.sync_copy(acca.at[pl.ds(sid * RZ, RZ)],
                        oa_hbm.at[pl.ds(cid * N_ACC + sid * RZ, RZ)])
        pltpu.sync_copy(accb.at[pl.ds(sid * RZ, RZ)],
                        ob_hbm.at[pl.ds(cid * N_ACC + sid * RZ, RZ),
                                  pl.ds(0, DB)])

    return agg


GRID = 16
R = N_ACC // GRID


def _dot_t(a, w):
    return lax.dot_general(a, w, (((1,), (1,)), ((), ())),
                           preferred_element_type=_f32)


def _spec(w, off=0):
    return pl.BlockSpec((R, w), lambda i, o=off: (i + o, 0))


def _wspec(r, c):
    return pl.BlockSpec((r, c), lambda i: (0, 0))


def _first_layer(aa, ab, x, wl, bl, wr):
    def body(a0, a1, c0, c1, x_ref, wl_ref, bl_ref, wr_ref,
             ha_ref, hb_ref, inv_ref):
        s = a0[...] + a1[...]
        inv = 1.0 / jnp.maximum(c0[:, 0:1] + c1[:, 0:1], 1.0)
        z = _dot_t(s * inv, wl_ref[...])
        z = z + bl_ref[0:1, :] + _dot_t(x_ref[...], wr_ref[...])
        z = jnp.maximum(z, 0.0)
        ha_ref[...] = z[:, :DA]
        hb_ref[...] = z[:, DA:]
        inv_ref[...] = jnp.broadcast_to(inv, (R, 16))

    return pl.pallas_call(
        body,
        grid=(GRID,),
        in_specs=[
            _spec(DA), _spec(DA, GRID), _spec(DA), _spec(DA, GRID),
            _spec(DA), _wspec(160, 128), _wspec(8, 160), _wspec(160, 128),
        ],
        out_specs=[_spec(DA), _spec(DB), _spec(16)],
        out_shape=[
            jax.ShapeDtypeStruct((N_ACC, DA), _f32),
            jax.ShapeDtypeStruct((N_ACC, DB), _f32),
            jax.ShapeDtypeStruct((N_ACC, 16), _f32),
        ],
    )(aa, aa, ab, ab, x, wl, bl, wr)


def _layer(aa, ab, inv, ha, hb, wl, bl, wr, relu):
    dout = wl.shape[0]

    def body(a0, a1, b0, b1, inv_ref, ha_ref, hb_ref,
             wl_ref, bl_ref, wr_ref, *outs):
        s = jnp.concatenate(
            [a0[...] + a1[...], b0[:, :DB] + b1[:, :DB]], axis=1)
        s = s * inv_ref[:, 0:1]
        h = jnp.concatenate([ha_ref[...], hb_ref[...]], axis=1)
        z = _dot_t(s, wl_ref[...])
        z = z + bl_ref[0:1, :] + _dot_t(h, wr_ref[...])
        if relu:
            z = jnp.maximum(z, 0.0)
        if len(outs) == 2:
            outs[0][...] = z[:, :DA]
            outs[1][...] = z[:, DA:]
        else:
            outs[0][...] = z

    if dout == 160:
        out_specs = [_spec(DA), _spec(DB)]
        out_shape = [jax.ShapeDtypeStruct((N_ACC, DA), _f32),
                     jax.ShapeDtypeStruct((N_ACC, DB), _f32)]
    else:
        out_specs = [_spec(dout)]
        out_shape = [jax.ShapeDtypeStruct((N_ACC, dout), _f32)]

    return pl.pallas_call(
        body,
        grid=(GRID,),
        in_specs=[
            _spec(DA), _spec(DA, GRID), _spec(DA), _spec(DA, GRID),
            _spec(16), _spec(DA), _spec(DB),
            _wspec(dout, 160), _wspec(8, dout), _wspec(dout, 160),
        ],
        out_specs=out_specs,
        out_shape=out_shape,
    )(aa, aa, ab, ab, inv, ha, hb, wl, bl, wr)


def _pool(h, batch2d):
    def body(h_ref, b_ref, out_ref):
        hv = h_ref[...]
        b = b_ref[0:1, :]
        gid = lax.broadcasted_iota(jnp.int32, (G, N_ACC), 0)
        onehot = (b == gid).astype(_f32)
        pooled = lax.dot_general(onehot, hv, (((1,), (0,)), ((), ())),
                                 preferred_element_type=_f32)
        gcnt = jnp.sum(onehot, axis=1, keepdims=True)
        out_ref[...] = pooled / jnp.maximum(gcnt, 1.0)

    return pl.pallas_call(
        body,
        grid=(1,),
        in_specs=[
            pl.BlockSpec((N_ACC, 128), lambda i: (0, 0)),
            pl.BlockSpec((8, N_ACC), lambda i: (0, 0)),
        ],
        out_specs=pl.BlockSpec((G, 128), lambda i: (0, 0)),
        out_shape=jax.ShapeDtypeStruct((G, 128), _f32),
    )(h, batch2d)


def _pad(a, r, c):
    return jnp.zeros((r, c), _f32).at[: a.shape[0], : a.shape[1]].set(a)


def kernel(x, edge_index, batch, W_l_in, b_l_in, W_r_in, W_l_mid, b_l_mid,
           W_r_mid, W_l_out, b_l_out, W_r_out):
    src = jnp.pad(edge_index[0].astype(jnp.int32).reshape(NW, EPW),
                  ((0, 0), (0, EPAD))).reshape(NW, CPW, K)
    dst = jnp.pad(edge_index[1].astype(jnp.int32).reshape(NW, EPW),
                  ((0, 0), (0, EPAD)),
                  constant_values=N).reshape(NW, CPW, K)
    sd = jnp.stack([src, dst], axis=2)

    xp = _pad(x, N_ACC, DA)
    za = jnp.zeros((N_ACC, DA), _f32)
    zb = jnp.zeros((N_ACC, DB), _f32)

    wl_in = _pad(W_l_in, 160, 128)
    wr_in = _pad(W_r_in, 160, 128)
    bl_in = jnp.broadcast_to(_pad(b_l_in[None, :], 1, 160), (8, 160))
    wl_mid = jnp.zeros((NMID, 160, 160), _f32).at[:, :150, :150].set(W_l_mid)
    wr_mid = jnp.zeros((NMID, 160, 160), _f32).at[:, :150, :150].set(W_r_mid)
    bl_mid = jnp.zeros((NMID, 8, 160), _f32).at[:, :, :150].set(
        jnp.broadcast_to(b_l_mid[:, None, :], (NMID, 8, 150)))
    wl_out = _pad(W_l_out, 128, 160)
    wr_out = _pad(W_r_out, 128, 160)
    bl_out = jnp.broadcast_to(_pad(b_l_out[None, :], 1, 128), (8, 128))

    batch2d = jnp.broadcast_to(
        jnp.pad(batch.astype(jnp.int32), (0, N_ACC - N),
                constant_values=G)[None, :], (8, N_ACC))

    agg_first = _make_agg(True)
    agg_mid = _make_agg(False)

    aa, ab = agg_first(xp, zb, sd, za, zb)
    ha, hb, inv = _first_layer(aa, ab, xp, wl_in, bl_in, wr_in)
    for i in range(NMID):
        aa, ab = agg_mid(ha, hb, sd, za, zb)
        ha, hb = _layer(aa, ab, inv, ha, hb,
                        wl_mid[i], bl_mid[i], wr_mid[i], True)
    aa, ab = agg_mid(ha, hb, sd, za, zb)
    h, = _layer(aa, ab, inv, ha, hb, wl_out, bl_out, wr_out, False)
    return _pool(h, batch2d)


# speedup vs baseline: 1.1716x; 1.1716x over previous
import functools

import jax
import jax.numpy as jnp
from jax import lax
from jax.experimental import pallas as pl
from jax.experimental.pallas import tpu as pltpu
from jax.experimental.pallas import tpu_sc as plsc

N = 10000
E = 320000
G = 64
NMID = 13
NC, NS = 2, 16
NW = NC * NS
K = 80
EPW = E // NW
CPW = -(-EPW // K)
EPAD = CPW * K - EPW
N_ACC = 10112
RZ = N_ACC // NS
DA = 128
DB = 32

_f32 = jnp.float32


@functools.lru_cache(maxsize=None)
def _make_agg(first):
    mesh = plsc.VectorSubcoreMesh(core_axis_name="c", subcore_axis_name="s")

    scratch = [
        pltpu.VMEM((K,), jnp.int32),
        pltpu.VMEM((K,), jnp.int32),
        pltpu.VMEM((K,), jnp.int32),
        pltpu.VMEM((K,), jnp.int32),
        pltpu.VMEM((K, DA), _f32),
        pltpu.VMEM((K, DB), _f32),
        pltpu.VMEM((K, DA), _f32),
        pltpu.VMEM((K, DB), _f32),
        pltpu.VMEM_SHARED((N_ACC, DA), _f32),
        pltpu.VMEM_SHARED((N_ACC, DB), _f32),
        pltpu.SemaphoreType.DMA,
        pltpu.SemaphoreType.DMA,
        pltpu.SemaphoreType.DMA,
        pltpu.SemaphoreType.DMA,
        pltpu.SemaphoreType.DMA,
        pltpu.SemaphoreType.DMA,
    ]

    @functools.partial(
        pl.kernel,
        out_type=[jax.ShapeDtypeStruct((NC * N_ACC, DA), _f32),
                  jax.ShapeDtypeStruct((NC * N_ACC, DA), _f32)],
        mesh=mesh,
        compiler_params=pltpu.CompilerParams(use_tc_tiling_on_sc=False),
        scratch_types=scratch,
    )
    def agg(ta_hbm, tb_hbm, src_hbm, dst_hbm, za_hbm, zb_hbm,
            oa_hbm, ob_hbm, s0, d0, s1, d1, bufa, bufa2, bufb,
            bufb2, acca, accb, sema, semb, ss0, ss1, sd0, sd1):
        cid = lax.axis_index("c")
        sid = lax.axis_index("s")
        wid = sid * NC + cid

        def fetch(hbm, c, v, sem):
            pltpu.async_copy(hbm.at[wid, c], v, sem)

        def wait_fetch(hbm, c, v, sem):
            pltpu.make_async_copy(hbm.at[wid, c], v, sem).wait()

        def gather(sv, buf, buf2, sem):
            pltpu.async_copy(ta_hbm.at[sv], buf, sem)
            if not first:
                pltpu.async_copy(tb_hbm.at[sv], buf2, sem)

        def wait_gather(sv, buf, buf2, sem):
            pltpu.make_async_copy(ta_hbm.at[sv], buf, sem).wait()
            if not first:
                pltpu.make_async_copy(tb_hbm.at[sv], buf2, sem).wait()

        def scatter(buf, buf2, dv):
            pltpu.sync_copy(buf, acca.at[dv], add=True)
            pltpu.sync_copy(buf2, accb.at[dv], add=True)

        if first:
            @pl.loop(0, K)
            def _(r):
                ones = jnp.full((16,), 1.0, _f32)
                bufa2[r, pl.ds(0, 16)] = ones
                bufa2[r, pl.ds(16, 16)] = ones
                bufb2[r, pl.ds(0, 16)] = ones
                bufb2[r, pl.ds(16, 16)] = ones

        fetch(src_hbm, 0, s0, ss0)
        fetch(dst_hbm, 0, d0, sd0)
        fetch(src_hbm, 1, s1, ss1)
        fetch(dst_hbm, 1, d1, sd1)
        pltpu.sync_copy(za_hbm.at[pl.ds(sid * RZ, RZ)],
                        acca.at[pl.ds(sid * RZ, RZ)])
        pltpu.sync_copy(zb_hbm.at[pl.ds(sid * RZ, RZ)],
                        accb.at[pl.ds(sid * RZ, RZ)])
        wait_fetch(src_hbm, 0, s0, ss0)
        gather(s0, bufa, bufa2, sema)
        plsc.subcore_barrier()

        @pl.loop(0, CPW - 1, step=2)
        def _(j):
            wait_fetch(src_hbm, j + 1, s1, ss1)
            gather(s1, bufb, bufb2, semb)
            wait_gather(s0, bufa, bufa2, sema)

            @pl.when(j + 2 < CPW)
            def _():
                fetch(src_hbm, j + 2, s0, ss0)

            wait_fetch(dst_hbm, j, d0, sd0)
            scatter(bufa, bufa2, d0)

            @pl.when(j + 2 < CPW)
            def _():
                fetch(dst_hbm, j + 2, d0, sd0)
                wait_fetch(src_hbm, j + 2, s0, ss0)
                gather(s0, bufa, bufa2, sema)

            wait_gather(s1, bufb, bufb2, semb)

            @pl.when(j + 3 < CPW)
            def _():
                fetch(src_hbm, j + 3, s1, ss1)

            wait_fetch(dst_hbm, j + 1, d1, sd1)
            scatter(bufb, bufb2, d1)

            @pl.when(j + 3 < CPW)
            def _():
                fetch(dst_hbm, j + 3, d1, sd1)

        wait_gather(s0, bufa, bufa2, sema)
        wait_fetch(dst_hbm, CPW - 1, d0, sd0)
        scatter(bufa, bufa2, d0)

        plsc.subcore_barrier()
        pltpu.sync_copy(acca.at[pl.ds(sid * RZ, RZ)],
                        oa_hbm.at[pl.ds(cid * N_ACC + sid * RZ, RZ)])
        pltpu.sync_copy(accb.at[pl.ds(sid * RZ, RZ)],
                        ob_hbm.at[pl.ds(cid * N_ACC + sid * RZ, RZ),
                                  pl.ds(0, DB)])

    return agg


GRID = 16
R = N_ACC // GRID


def _dot_t(a, w):
    return lax.dot_general(a, w, (((1,), (1,)), ((), ())),
                           preferred_element_type=_f32)


def _spec(w, off=0):
    return pl.BlockSpec((R, w), lambda i, o=off: (i + o, 0))


def _wspec(r, c):
    return pl.BlockSpec((r, c), lambda i: (0, 0))


def _first_layer(aa, ab, x, wl, bl, wr):
    def body(a0, a1, c0, c1, x_ref, wl_ref, bl_ref, wr_ref,
             ha_ref, hb_ref, inv_ref):
        s = a0[...] + a1[...]
        inv = 1.0 / jnp.maximum(c0[:, 0:1] + c1[:, 0:1], 1.0)
        z = _dot_t(s * inv, wl_ref[...])
        z = z + bl_ref[0:1, :] + _dot_t(x_ref[...], wr_ref[...])
        z = jnp.maximum(z, 0.0)
        ha_ref[...] = z[:, :DA]
        hb_ref[...] = z[:, DA:]
        inv_ref[...] = jnp.broadcast_to(inv, (R, 16))

    return pl.pallas_call(
        body,
        grid=(GRID,),
        in_specs=[
            _spec(DA), _spec(DA, GRID), _spec(DA), _spec(DA, GRID),
            _spec(DA), _wspec(160, 128), _wspec(8, 160), _wspec(160, 128),
        ],
        out_specs=[_spec(DA), _spec(DB), _spec(16)],
        out_shape=[
            jax.ShapeDtypeStruct((N_ACC, DA), _f32),
            jax.ShapeDtypeStruct((N_ACC, DB), _f32),
            jax.ShapeDtypeStruct((N_ACC, 16), _f32),
        ],
    )(aa, aa, ab, ab, x, wl, bl, wr)


def _layer(aa, ab, inv, ha, hb, wl, bl, wr, relu):
    dout = wl.shape[0]

    def body(a0, a1, b0, b1, inv_ref, ha_ref, hb_ref,
             wl_ref, bl_ref, wr_ref, *outs):
        s = jnp.concatenate(
            [a0[...] + a1[...], b0[:, :DB] + b1[:, :DB]], axis=1)
        s = s * inv_ref[:, 0:1]
        h = jnp.concatenate([ha_ref[...], hb_ref[...]], axis=1)
        z = _dot_t(s, wl_ref[...])
        z = z + bl_ref[0:1, :] + _dot_t(h, wr_ref[...])
        if relu:
            z = jnp.maximum(z, 0.0)
        if len(outs) == 2:
            outs[0][...] = z[:, :DA]
            outs[1][...] = z[:, DA:]
        else:
            outs[0][...] = z

    if dout == 160:
        out_specs = [_spec(DA), _spec(DB)]
        out_shape = [jax.ShapeDtypeStruct((N_ACC, DA), _f32),
                     jax.ShapeDtypeStruct((N_ACC, DB), _f32)]
    else:
        out_specs = [_spec(dout)]
        out_shape = [jax.ShapeDtypeStruct((N_ACC, dout), _f32)]

    return pl.pallas_call(
        body,
        grid=(GRID,),
        in_specs=[
            _spec(DA), _spec(DA, GRID), _spec(DA), _spec(DA, GRID),
            _spec(16), _spec(DA), _spec(DB),
            _wspec(dout, 160), _wspec(8, dout), _wspec(dout, 160),
        ],
        out_specs=out_specs,
        out_shape=out_shape,
    )(aa, aa, ab, ab, inv, ha, hb, wl, bl, wr)


def _pool(h, batch2d):
    def body(h_ref, b_ref, out_ref):
        hv = h_ref[...]
        b = b_ref[0:1, :]
        gid = lax.broadcasted_iota(jnp.int32, (G, N_ACC), 0)
        onehot = (b == gid).astype(_f32)
        pooled = lax.dot_general(onehot, hv, (((1,), (0,)), ((), ())),
                                 preferred_element_type=_f32)
        gcnt = jnp.sum(onehot, axis=1, keepdims=True)
        out_ref[...] = pooled / jnp.maximum(gcnt, 1.0)

    return pl.pallas_call(
        body,
        grid=(1,),
        in_specs=[
            pl.BlockSpec((N_ACC, 128), lambda i: (0, 0)),
            pl.BlockSpec((8, N_ACC), lambda i: (0, 0)),
        ],
        out_specs=pl.BlockSpec((G, 128), lambda i: (0, 0)),
        out_shape=jax.ShapeDtypeStruct((G, 128), _f32),
    )(h, batch2d)


def _pad(a, r, c):
    return jnp.zeros((r, c), _f32).at[: a.shape[0], : a.shape[1]].set(a)


def kernel(x, edge_index, batch, W_l_in, b_l_in, W_r_in, W_l_mid, b_l_mid,
           W_r_mid, W_l_out, b_l_out, W_r_out):
    src = jnp.pad(edge_index[0].astype(jnp.int32).reshape(NW, EPW),
                  ((0, 0), (0, EPAD))).reshape(NW, CPW, K)
    dst = jnp.pad(edge_index[1].astype(jnp.int32).reshape(NW, EPW),
                  ((0, 0), (0, EPAD)),
                  constant_values=N).reshape(NW, CPW, K)

    xp = _pad(x, N_ACC, DA)
    za = jnp.zeros((N_ACC, DA), _f32)
    zb = jnp.zeros((N_ACC, DB), _f32)

    wl_in = _pad(W_l_in, 160, 128)
    wr_in = _pad(W_r_in, 160, 128)
    bl_in = jnp.broadcast_to(_pad(b_l_in[None, :], 1, 160), (8, 160))
    wl_mid = jnp.zeros((NMID, 160, 160), _f32).at[:, :150, :150].set(W_l_mid)
    wr_mid = jnp.zeros((NMID, 160, 160), _f32).at[:, :150, :150].set(W_r_mid)
    bl_mid = jnp.zeros((NMID, 8, 160), _f32).at[:, :, :150].set(
        jnp.broadcast_to(b_l_mid[:, None, :], (NMID, 8, 150)))
    wl_out = _pad(W_l_out, 128, 160)
    wr_out = _pad(W_r_out, 128, 160)
    bl_out = jnp.broadcast_to(_pad(b_l_out[None, :], 1, 128), (8, 128))

    batch2d = jnp.broadcast_to(
        jnp.pad(batch.astype(jnp.int32), (0, N_ACC - N),
                constant_values=G)[None, :], (8, N_ACC))

    agg_first = _make_agg(True)
    agg_mid = _make_agg(False)

    aa, ab = agg_first(xp, zb, src, dst, za, zb)
    ha, hb, inv = _first_layer(aa, ab, xp, wl_in, bl_in, wr_in)
    for i in range(NMID):
        aa, ab = agg_mid(ha, hb, src, dst, za, zb)
        ha, hb = _layer(aa, ab, inv, ha, hb,
                        wl_mid[i], bl_mid[i], wr_mid[i], True)
    aa, ab = agg_mid(ha, hb, src, dst, za, zb)
    h, = _layer(aa, ab, inv, ha, hb, wl_out, bl_out, wr_out, False)
    return _pool(h, batch2d)


# speedup vs baseline: 1.1825x; 1.0092x over previous
import functools

import jax
import jax.numpy as jnp
from jax import lax
from jax.experimental import pallas as pl
from jax.experimental.pallas import tpu as pltpu
from jax.experimental.pallas import tpu_sc as plsc

N = 10000
E = 320000
G = 64
NMID = 13
NC, NS = 2, 16
NW = NC * NS
K = 80
EPW = E // NW
CPW = -(-EPW // K)
EPAD = CPW * K - EPW
N_ACC = 10112
RZ = N_ACC // NS
DA = 128
DB = 32

_f32 = jnp.float32


@functools.lru_cache(maxsize=None)
def _make_agg(first):
    mesh = plsc.VectorSubcoreMesh(core_axis_name="c", subcore_axis_name="s")

    scratch = [
        pltpu.VMEM((K,), jnp.int32),
        pltpu.VMEM((K,), jnp.int32),
        pltpu.VMEM((K,), jnp.int32),
        pltpu.VMEM((K,), jnp.int32),
        pltpu.VMEM((K, DA), _f32),
        pltpu.VMEM((K, DB), _f32),
        pltpu.VMEM((K, DA), _f32),
        pltpu.VMEM((K, DB), _f32),
        pltpu.VMEM_SHARED((N_ACC, DA), _f32),
        pltpu.VMEM_SHARED((N_ACC, DB), _f32),
        pltpu.SemaphoreType.DMA,
        pltpu.SemaphoreType.DMA,
        pltpu.SemaphoreType.DMA,
        pltpu.SemaphoreType.DMA,
        pltpu.SemaphoreType.DMA,
        pltpu.SemaphoreType.DMA,
        pltpu.SemaphoreType.DMA,
    ]

    @functools.partial(
        pl.kernel,
        out_type=[jax.ShapeDtypeStruct((NC * N_ACC, DA), _f32),
                  jax.ShapeDtypeStruct((NC * N_ACC, DA), _f32)],
        mesh=mesh,
        compiler_params=pltpu.CompilerParams(use_tc_tiling_on_sc=False),
        scratch_types=scratch,
    )
    def agg(ta_hbm, tb_hbm, src_hbm, dst_hbm, za_hbm, zb_hbm,
            oa_hbm, ob_hbm, s0, d0, s1, d1, bufa, bufa2, bufb,
            bufb2, acca, accb, sema, semb, ss0, ss1, sd0, sd1, sesc):
        cid = lax.axis_index("c")
        sid = lax.axis_index("s")
        wid = sid * NC + cid

        def fetch(hbm, c, v, sem):
            pltpu.async_copy(hbm.at[wid, c], v, sem)

        def wait_fetch(hbm, c, v, sem):
            pltpu.make_async_copy(hbm.at[wid, c], v, sem).wait()

        def gather(sv, buf, buf2, sem):
            pltpu.async_copy(ta_hbm.at[sv], buf, sem)
            if not first:
                pltpu.async_copy(tb_hbm.at[sv], buf2, sem)

        def wait_gather(sv, buf, buf2, sem):
            pltpu.make_async_copy(ta_hbm.at[sv], buf, sem).wait()
            if not first:
                pltpu.make_async_copy(tb_hbm.at[sv], buf2, sem).wait()

        def scatter(buf, buf2, dv):
            cp = pltpu.async_copy(buf, acca.at[dv], sesc, add=True)
            pltpu.sync_copy(buf2, accb.at[dv], add=True)
            cp.wait()

        if first:
            @pl.loop(0, K)
            def _(r):
                ones = jnp.full((16,), 1.0, _f32)
                bufa2[r, pl.ds(0, 16)] = ones
                bufa2[r, pl.ds(16, 16)] = ones
                bufb2[r, pl.ds(0, 16)] = ones
                bufb2[r, pl.ds(16, 16)] = ones

        fetch(src_hbm, 0, s0, ss0)
        fetch(dst_hbm, 0, d0, sd0)
        fetch(src_hbm, 1, s1, ss1)
        fetch(dst_hbm, 1, d1, sd1)
        pltpu.sync_copy(za_hbm.at[pl.ds(sid * RZ, RZ)],
                        acca.at[pl.ds(sid * RZ, RZ)])
        pltpu.sync_copy(zb_hbm.at[pl.ds(sid * RZ, RZ)],
                        accb.at[pl.ds(sid * RZ, RZ)])
        wait_fetch(src_hbm, 0, s0, ss0)
        gather(s0, bufa, bufa2, sema)
        plsc.subcore_barrier()

        @pl.loop(0, CPW - 1, step=2)
        def _(j):
            wait_fetch(src_hbm, j + 1, s1, ss1)
            gather(s1, bufb, bufb2, semb)
            wait_gather(s0, bufa, bufa2, sema)

            @pl.when(j + 2 < CPW)
            def _():
                fetch(src_hbm, j + 2, s0, ss0)

            wait_fetch(dst_hbm, j, d0, sd0)
            scatter(bufa, bufa2, d0)

            @pl.when(j + 2 < CPW)
            def _():
                fetch(dst_hbm, j + 2, d0, sd0)
                wait_fetch(src_hbm, j + 2, s0, ss0)
                gather(s0, bufa, bufa2, sema)

            wait_gather(s1, bufb, bufb2, semb)

            @pl.when(j + 3 < CPW)
            def _():
                fetch(src_hbm, j + 3, s1, ss1)

            wait_fetch(dst_hbm, j + 1, d1, sd1)
            scatter(bufb, bufb2, d1)

            @pl.when(j + 3 < CPW)
            def _():
                fetch(dst_hbm, j + 3, d1, sd1)

        wait_gather(s0, bufa, bufa2, sema)
        wait_fetch(dst_hbm, CPW - 1, d0, sd0)
        scatter(bufa, bufa2, d0)

        plsc.subcore_barrier()
        pltpu.sync_copy(acca.at[pl.ds(sid * RZ, RZ)],
                        oa_hbm.at[pl.ds(cid * N_ACC + sid * RZ, RZ)])
        pltpu.sync_copy(accb.at[pl.ds(sid * RZ, RZ)],
                        ob_hbm.at[pl.ds(cid * N_ACC + sid * RZ, RZ),
                                  pl.ds(0, DB)])

    return agg


GRID = 16
R = N_ACC // GRID


def _dot_t(a, w):
    return lax.dot_general(a, w, (((1,), (1,)), ((), ())),
                           preferred_element_type=_f32)


def _spec(w, off=0):
    return pl.BlockSpec((R, w), lambda i, o=off: (i + o, 0))


def _wspec(r, c):
    return pl.BlockSpec((r, c), lambda i: (0, 0))


def _first_layer(aa, ab, x, wl, bl, wr):
    def body(a0, a1, c0, c1, x_ref, wl_ref, bl_ref, wr_ref,
             ha_ref, hb_ref, inv_ref):
        s = a0[...] + a1[...]
        inv = 1.0 / jnp.maximum(c0[:, 0:1] + c1[:, 0:1], 1.0)
        z = _dot_t(s * inv, wl_ref[...])
        z = z + bl_ref[0:1, :] + _dot_t(x_ref[...], wr_ref[...])
        z = jnp.maximum(z, 0.0)
        ha_ref[...] = z[:, :DA]
        hb_ref[...] = z[:, DA:]
        inv_ref[...] = jnp.broadcast_to(inv, (R, 16))

    return pl.pallas_call(
        body,
        grid=(GRID,),
        in_specs=[
            _spec(DA), _spec(DA, GRID), _spec(DA), _spec(DA, GRID),
            _spec(DA), _wspec(160, 128), _wspec(8, 160), _wspec(160, 128),
        ],
        out_specs=[_spec(DA), _spec(DB), _spec(16)],
        out_shape=[
            jax.ShapeDtypeStruct((N_ACC, DA), _f32),
            jax.ShapeDtypeStruct((N_ACC, DB), _f32),
            jax.ShapeDtypeStruct((N_ACC, 16), _f32),
        ],
    )(aa, aa, ab, ab, x, wl, bl, wr)


def _layer(aa, ab, inv, ha, hb, wl, bl, wr, relu):
    dout = wl.shape[0]

    def body(a0, a1, b0, b1, inv_ref, ha_ref, hb_ref,
             wl_ref, bl_ref, wr_ref, *outs):
        s = jnp.concatenate(
            [a0[...] + a1[...], b0[:, :DB] + b1[:, :DB]], axis=1)
        s = s * inv_ref[:, 0:1]
        h = jnp.concatenate([ha_ref[...], hb_ref[...]], axis=1)
        z = _dot_t(s, wl_ref[...])
        z = z + bl_ref[0:1, :] + _dot_t(h, wr_ref[...])
        if relu:
            z = jnp.maximum(z, 0.0)
        if len(outs) == 2:
            outs[0][...] = z[:, :DA]
            outs[1][...] = z[:, DA:]
        else:
            outs[0][...] = z

    if dout == 160:
        out_specs = [_spec(DA), _spec(DB)]
        out_shape = [jax.ShapeDtypeStruct((N_ACC, DA), _f32),
                     jax.ShapeDtypeStruct((N_ACC, DB), _f32)]
    else:
        out_specs = [_spec(dout)]
        out_shape = [jax.ShapeDtypeStruct((N_ACC, dout), _f32)]

    return pl.pallas_call(
        body,
        grid=(GRID,),
        in_specs=[
            _spec(DA), _spec(DA, GRID), _spec(DA), _spec(DA, GRID),
            _spec(16), _spec(DA), _spec(DB),
            _wspec(dout, 160), _wspec(8, dout), _wspec(dout, 160),
        ],
        out_specs=out_specs,
        out_shape=out_shape,
    )(aa, aa, ab, ab, inv, ha, hb, wl, bl, wr)


def _pool(h, batch2d):
    def body(h_ref, b_ref, out_ref):
        hv = h_ref[...]
        b = b_ref[0:1, :]
        gid = lax.broadcasted_iota(jnp.int32, (G, N_ACC), 0)
        onehot = (b == gid).astype(_f32)
        pooled = lax.dot_general(onehot, hv, (((1,), (0,)), ((), ())),
                                 preferred_element_type=_f32)
        gcnt = jnp.sum(onehot, axis=1, keepdims=True)
        out_ref[...] = pooled / jnp.maximum(gcnt, 1.0)

    return pl.pallas_call(
        body,
        grid=(1,),
        in_specs=[
            pl.BlockSpec((N_ACC, 128), lambda i: (0, 0)),
            pl.BlockSpec((8, N_ACC), lambda i: (0, 0)),
        ],
        out_specs=pl.BlockSpec((G, 128), lambda i: (0, 0)),
        out_shape=jax.ShapeDtypeStruct((G, 128), _f32),
    )(h, batch2d)


def _pad(a, r, c):
    return jnp.zeros((r, c), _f32).at[: a.shape[0], : a.shape[1]].set(a)


def kernel(x, edge_index, batch, W_l_in, b_l_in, W_r_in, W_l_mid, b_l_mid,
           W_r_mid, W_l_out, b_l_out, W_r_out):
    src = jnp.pad(edge_index[0].astype(jnp.int32).reshape(NW, EPW),
                  ((0, 0), (0, EPAD))).reshape(NW, CPW, K)
    dst = jnp.pad(edge_index[1].astype(jnp.int32).reshape(NW, EPW),
                  ((0, 0), (0, EPAD)),
                  constant_values=N).reshape(NW, CPW, K)

    xp = _pad(x, N_ACC, DA)
    za = jnp.zeros((N_ACC, DA), _f32)
    zb = jnp.zeros((N_ACC, DB), _f32)

    wl_in = _pad(W_l_in, 160, 128)
    wr_in = _pad(W_r_in, 160, 128)
    bl_in = jnp.broadcast_to(_pad(b_l_in[None, :], 1, 160), (8, 160))
    wl_mid = jnp.zeros((NMID, 160, 160), _f32).at[:, :150, :150].set(W_l_mid)
    wr_mid = jnp.zeros((NMID, 160, 160), _f32).at[:, :150, :150].set(W_r_mid)
    bl_mid = jnp.zeros((NMID, 8, 160), _f32).at[:, :, :150].set(
        jnp.broadcast_to(b_l_mid[:, None, :], (NMID, 8, 150)))
    wl_out = _pad(W_l_out, 128, 160)
    wr_out = _pad(W_r_out, 128, 160)
    bl_out = jnp.broadcast_to(_pad(b_l_out[None, :], 1, 128), (8, 128))

    batch2d = jnp.broadcast_to(
        jnp.pad(batch.astype(jnp.int32), (0, N_ACC - N),
                constant_values=G)[None, :], (8, N_ACC))

    agg_first = _make_agg(True)
    agg_mid = _make_agg(False)

    aa, ab = agg_first(xp, zb, src, dst, za, zb)
    ha, hb, inv = _first_layer(aa, ab, xp, wl_in, bl_in, wr_in)
    for i in range(NMID):
        aa, ab = agg_mid(ha, hb, src, dst, za, zb)
        ha, hb = _layer(aa, ab, inv, ha, hb,
                        wl_mid[i], bl_mid[i], wr_mid[i], True)
    aa, ab = agg_mid(ha, hb, src, dst, za, zb)
    h, = _layer(aa, ab, inv, ha, hb, wl_out, bl_out, wr_out, False)
    return _pool(h, batch2d)


# speedup vs baseline: 1.2095x; 1.0229x over previous
import functools

import jax
import jax.numpy as jnp
from jax import lax
from jax.experimental import pallas as pl
from jax.experimental.pallas import tpu as pltpu
from jax.experimental.pallas import tpu_sc as plsc

N = 10000
E = 320000
G = 64
NMID = 13
NC, NS = 2, 16
NW = NC * NS
K = 80
EPW = E // NW
CPW = -(-EPW // K)
EPAD = CPW * K - EPW
N_ACC = 10112
RZ = N_ACC // NS
DA = 128
DB = 32

_f32 = jnp.float32


@functools.lru_cache(maxsize=None)
def _make_agg(first):
    mesh = plsc.VectorSubcoreMesh(core_axis_name="c", subcore_axis_name="s")

    scratch = [
        pltpu.VMEM((K,), jnp.int32),
        pltpu.VMEM((K,), jnp.int32),
        pltpu.VMEM((K,), jnp.int32),
        pltpu.VMEM((K,), jnp.int32),
        pltpu.VMEM((K, DA), _f32),
        pltpu.VMEM((K, DB), _f32),
        pltpu.VMEM((K, DA), _f32),
        pltpu.VMEM((K, DB), _f32),
        pltpu.VMEM_SHARED((N_ACC, DA), _f32),
        pltpu.VMEM_SHARED((N_ACC, DB), _f32),
        pltpu.SemaphoreType.DMA,
        pltpu.SemaphoreType.DMA,
        pltpu.SemaphoreType.DMA,
        pltpu.SemaphoreType.DMA,
        pltpu.SemaphoreType.DMA,
        pltpu.SemaphoreType.DMA,
        pltpu.SemaphoreType.DMA,
    ]

    @functools.partial(
        pl.kernel,
        out_type=[jax.ShapeDtypeStruct((NC * N_ACC, DA), _f32),
                  jax.ShapeDtypeStruct((NC * N_ACC, DA), _f32)],
        mesh=mesh,
        compiler_params=pltpu.CompilerParams(use_tc_tiling_on_sc=False),
        scratch_types=scratch,
    )
    def agg(ta_hbm, tb_hbm, src_hbm, dst_hbm, za_hbm, zb_hbm,
            oa_hbm, ob_hbm, s0, d0, s1, d1, bufa, bufa2, bufb,
            bufb2, acca, accb, sema, semb, ss0, ss1, sd0, sd1, sesc):
        cid = lax.axis_index("c")
        sid = lax.axis_index("s")
        wid = sid * NC + cid

        def fetch(hbm, c, v, sem):
            pltpu.async_copy(hbm.at[wid, c], v, sem)

        def wait_fetch(hbm, c, v, sem):
            pltpu.make_async_copy(hbm.at[wid, c], v, sem).wait()

        def gather(sv, buf, buf2, sem):
            pltpu.async_copy(ta_hbm.at[sv], buf, sem)
            if not first:
                pltpu.async_copy(tb_hbm.at[sv], buf2, sem)

        def wait_gather(sv, buf, buf2, sem):
            pltpu.make_async_copy(ta_hbm.at[sv], buf, sem).wait()
            if not first:
                pltpu.make_async_copy(tb_hbm.at[sv], buf2, sem).wait()

        def scatter(buf, buf2, dv):
            cp = pltpu.async_copy(buf, acca.at[dv], sesc, add=True)
            pltpu.sync_copy(buf2, accb.at[dv], add=True)
            cp.wait()

        if first:
            @pl.loop(0, K)
            def _(r):
                ones = jnp.full((16,), 1.0, _f32)
                bufa2[r, pl.ds(0, 16)] = ones
                bufa2[r, pl.ds(16, 16)] = ones
                bufb2[r, pl.ds(0, 16)] = ones
                bufb2[r, pl.ds(16, 16)] = ones

        fetch(src_hbm, 0, s0, ss0)
        fetch(dst_hbm, 0, d0, sd0)
        fetch(src_hbm, 1, s1, ss1)
        fetch(dst_hbm, 1, d1, sd1)
        pltpu.sync_copy(za_hbm.at[pl.ds(sid * RZ, RZ)],
                        acca.at[pl.ds(sid * RZ, RZ)])
        pltpu.sync_copy(zb_hbm.at[pl.ds(sid * RZ, RZ)],
                        accb.at[pl.ds(sid * RZ, RZ)])
        wait_fetch(src_hbm, 0, s0, ss0)
        gather(s0, bufa, bufa2, sema)
        plsc.subcore_barrier()

        @pl.loop(0, CPW - 1, step=2)
        def _(j):
            wait_fetch(src_hbm, j + 1, s1, ss1)
            gather(s1, bufb, bufb2, semb)
            wait_gather(s0, bufa, bufa2, sema)

            @pl.when(j + 2 < CPW)
            def _():
                fetch(src_hbm, j + 2, s0, ss0)

            wait_fetch(dst_hbm, j, d0, sd0)
            scatter(bufa, bufa2, d0)

            @pl.when(j + 2 < CPW)
            def _():
                fetch(dst_hbm, j + 2, d0, sd0)
                wait_fetch(src_hbm, j + 2, s0, ss0)
                gather(s0, bufa, bufa2, sema)

            wait_gather(s1, bufb, bufb2, semb)

            @pl.when(j + 3 < CPW)
            def _():
                fetch(src_hbm, j + 3, s1, ss1)

            wait_fetch(dst_hbm, j + 1, d1, sd1)
            scatter(bufb, bufb2, d1)

            @pl.when(j + 3 < CPW)
            def _():
                fetch(dst_hbm, j + 3, d1, sd1)

        wait_gather(s0, bufa, bufa2, sema)
        wait_fetch(dst_hbm, CPW - 1, d0, sd0)
        scatter(bufa, bufa2, d0)

        plsc.subcore_barrier()
        pltpu.sync_copy(acca.at[pl.ds(sid * RZ, RZ)],
                        oa_hbm.at[pl.ds(cid * N_ACC + sid * RZ, RZ)])
        pltpu.sync_copy(accb.at[pl.ds(sid * RZ, RZ)],
                        ob_hbm.at[pl.ds(cid * N_ACC + sid * RZ, RZ),
                                  pl.ds(0, DB)])

    return agg


GRID = 8
R = N_ACC // GRID


def _dot_t(a, w):
    return lax.dot_general(a, w, (((1,), (1,)), ((), ())),
                           preferred_element_type=_f32)


def _spec(w, off=0):
    return pl.BlockSpec((R, w), lambda i, o=off: (i + o, 0))


def _wspec(r, c):
    return pl.BlockSpec((r, c), lambda i: (0, 0))


def _first_layer(aa, ab, x, wl, bl, wr):
    def body(a0, a1, c0, c1, x_ref, wl_ref, bl_ref, wr_ref,
             ha_ref, hb_ref, inv_ref):
        s = a0[...] + a1[...]
        inv = 1.0 / jnp.maximum(c0[:, 0:1] + c1[:, 0:1], 1.0)
        z = _dot_t(s * inv, wl_ref[...])
        z = z + bl_ref[0:1, :] + _dot_t(x_ref[...], wr_ref[...])
        z = jnp.maximum(z, 0.0)
        ha_ref[...] = z[:, :DA]
        hb_ref[...] = z[:, DA:]
        inv_ref[...] = jnp.broadcast_to(inv, (R, 16))

    return pl.pallas_call(
        body,
        grid=(GRID,),
        in_specs=[
            _spec(DA), _spec(DA, GRID), _spec(DA), _spec(DA, GRID),
            _spec(DA), _wspec(160, 128), _wspec(8, 160), _wspec(160, 128),
        ],
        out_specs=[_spec(DA), _spec(DB), _spec(16)],
        out_shape=[
            jax.ShapeDtypeStruct((N_ACC, DA), _f32),
            jax.ShapeDtypeStruct((N_ACC, DB), _f32),
            jax.ShapeDtypeStruct((N_ACC, 16), _f32),
        ],
    )(aa, aa, ab, ab, x, wl, bl, wr)


def _layer(aa, ab, inv, ha, hb, wl, bl, wr, relu):
    dout = wl.shape[0]

    def body(a0, a1, b0, b1, inv_ref, ha_ref, hb_ref,
             wl_ref, bl_ref, wr_ref, *outs):
        s = jnp.concatenate(
            [a0[...] + a1[...], b0[:, :DB] + b1[:, :DB]], axis=1)
        s = s * inv_ref[:, 0:1]
        h = jnp.concatenate([ha_ref[...], hb_ref[...]], axis=1)
        z = _dot_t(s, wl_ref[...])
        z = z + bl_ref[0:1, :] + _dot_t(h, wr_ref[...])
        if relu:
            z = jnp.maximum(z, 0.0)
        if len(outs) == 2:
            outs[0][...] = z[:, :DA]
            outs[1][...] = z[:, DA:]
        else:
            outs[0][...] = z

    if dout == 160:
        out_specs = [_spec(DA), _spec(DB)]
        out_shape = [jax.ShapeDtypeStruct((N_ACC, DA), _f32),
                     jax.ShapeDtypeStruct((N_ACC, DB), _f32)]
    else:
        out_specs = [_spec(dout)]
        out_shape = [jax.ShapeDtypeStruct((N_ACC, dout), _f32)]

    return pl.pallas_call(
        body,
        grid=(GRID,),
        in_specs=[
            _spec(DA), _spec(DA, GRID), _spec(DA), _spec(DA, GRID),
            _spec(16), _spec(DA), _spec(DB),
            _wspec(dout, 160), _wspec(8, dout), _wspec(dout, 160),
        ],
        out_specs=out_specs,
        out_shape=out_shape,
    )(aa, aa, ab, ab, inv, ha, hb, wl, bl, wr)


def _pool(h, batch2d):
    def body(h_ref, b_ref, out_ref):
        hv = h_ref[...]
        b = b_ref[0:1, :]
        gid = lax.broadcasted_iota(jnp.int32, (G, N_ACC), 0)
        onehot = (b == gid).astype(_f32)
        pooled = lax.dot_general(onehot, hv, (((1,), (0,)), ((), ())),
                                 preferred_element_type=_f32)
        gcnt = jnp.sum(onehot, axis=1, keepdims=True)
        out_ref[...] = pooled / jnp.maximum(gcnt, 1.0)

    return pl.pallas_call(
        body,
        grid=(1,),
        in_specs=[
            pl.BlockSpec((N_ACC, 128), lambda i: (0, 0)),
            pl.BlockSpec((8, N_ACC), lambda i: (0, 0)),
        ],
        out_specs=pl.BlockSpec((G, 128), lambda i: (0, 0)),
        out_shape=jax.ShapeDtypeStruct((G, 128), _f32),
    )(h, batch2d)


def _pad(a, r, c):
    return jnp.zeros((r, c), _f32).at[: a.shape[0], : a.shape[1]].set(a)


def kernel(x, edge_index, batch, W_l_in, b_l_in, W_r_in, W_l_mid, b_l_mid,
           W_r_mid, W_l_out, b_l_out, W_r_out):
    src = jnp.pad(edge_index[0].astype(jnp.int32).reshape(NW, EPW),
                  ((0, 0), (0, EPAD))).reshape(NW, CPW, K)
    dst = jnp.pad(edge_index[1].astype(jnp.int32).reshape(NW, EPW),
                  ((0, 0), (0, EPAD)),
                  constant_values=N).reshape(NW, CPW, K)

    xp = _pad(x, N_ACC, DA)
    za = jnp.zeros((N_ACC, DA), _f32)
    zb = jnp.zeros((N_ACC, DB), _f32)

    wl_in = _pad(W_l_in, 160, 128)
    wr_in = _pad(W_r_in, 160, 128)
    bl_in = jnp.broadcast_to(_pad(b_l_in[None, :], 1, 160), (8, 160))
    wl_mid = jnp.zeros((NMID, 160, 160), _f32).at[:, :150, :150].set(W_l_mid)
    wr_mid = jnp.zeros((NMID, 160, 160), _f32).at[:, :150, :150].set(W_r_mid)
    bl_mid = jnp.zeros((NMID, 8, 160), _f32).at[:, :, :150].set(
        jnp.broadcast_to(b_l_mid[:, None, :], (NMID, 8, 150)))
    wl_out = _pad(W_l_out, 128, 160)
    wr_out = _pad(W_r_out, 128, 160)
    bl_out = jnp.broadcast_to(_pad(b_l_out[None, :], 1, 128), (8, 128))

    batch2d = jnp.broadcast_to(
        jnp.pad(batch.astype(jnp.int32), (0, N_ACC - N),
                constant_values=G)[None, :], (8, N_ACC))

    agg_first = _make_agg(True)
    agg_mid = _make_agg(False)

    aa, ab = agg_first(xp, zb, src, dst, za, zb)
    ha, hb, inv = _first_layer(aa, ab, xp, wl_in, bl_in, wr_in)
    for i in range(NMID):
        aa, ab = agg_mid(ha, hb, src, dst, za, zb)
        ha, hb = _layer(aa, ab, inv, ha, hb,
                        wl_mid[i], bl_mid[i], wr_mid[i], True)
    aa, ab = agg_mid(ha, hb, src, dst, za, zb)
    h, = _layer(aa, ab, inv, ha, hb, wl_out, bl_out, wr_out, False)
    return _pool(h, batch2d)


# speedup vs baseline: 1.2133x; 1.0031x over previous
import functools

import jax
import jax.numpy as jnp
from jax import lax
from jax.experimental import pallas as pl
from jax.experimental.pallas import tpu as pltpu
from jax.experimental.pallas import tpu_sc as plsc

N = 10000
E = 320000
G = 64
NMID = 13
NC, NS = 2, 16
NW = NC * NS
K = 80
EPW = E // NW
CPW = -(-EPW // K)
EPAD = CPW * K - EPW
N_ACC = 10112
RZ = N_ACC // NS
DA = 128
DB = 32

_f32 = jnp.float32


@functools.lru_cache(maxsize=None)
def _make_agg(first):
    mesh = plsc.VectorSubcoreMesh(core_axis_name="c", subcore_axis_name="s")

    scratch = [
        pltpu.VMEM((K,), jnp.int32),
        pltpu.VMEM((K,), jnp.int32),
        pltpu.VMEM((K,), jnp.int32),
        pltpu.VMEM((K,), jnp.int32),
        pltpu.VMEM((K, DA), _f32),
        pltpu.VMEM((K, DB), _f32),
        pltpu.VMEM((K, DA), _f32),
        pltpu.VMEM((K, DB), _f32),
        pltpu.VMEM_SHARED((N_ACC, DA), _f32),
        pltpu.VMEM_SHARED((N_ACC, DB), _f32),
        pltpu.SemaphoreType.DMA,
        pltpu.SemaphoreType.DMA,
        pltpu.SemaphoreType.DMA,
        pltpu.SemaphoreType.DMA,
        pltpu.SemaphoreType.DMA,
        pltpu.SemaphoreType.DMA,
        pltpu.SemaphoreType.DMA,
    ]

    @functools.partial(
        pl.kernel,
        out_type=[jax.ShapeDtypeStruct((NC * N_ACC, DA), _f32),
                  jax.ShapeDtypeStruct((NC * N_ACC, DA), _f32)],
        mesh=mesh,
        compiler_params=pltpu.CompilerParams(use_tc_tiling_on_sc=False),
        scratch_types=scratch,
    )
    def agg(ta_hbm, tb_hbm, src_hbm, dst_hbm, za_hbm, zb_hbm,
            oa_hbm, ob_hbm, s0, d0, s1, d1, bufa, bufa2, bufb,
            bufb2, acca, accb, sema, semb, ss0, ss1, sd0, sd1, sesc):
        cid = lax.axis_index("c")
        sid = lax.axis_index("s")
        wid = sid * NC + cid

        def fetch(hbm, c, v, sem):
            pltpu.async_copy(hbm.at[wid, c], v, sem)

        def wait_fetch(hbm, c, v, sem):
            pltpu.make_async_copy(hbm.at[wid, c], v, sem).wait()

        def gather(sv, buf, buf2, sem):
            pltpu.async_copy(ta_hbm.at[sv], buf, sem)
            if not first:
                pltpu.async_copy(tb_hbm.at[sv], buf2, sem)

        def wait_gather(sv, buf, buf2, sem):
            pltpu.make_async_copy(ta_hbm.at[sv], buf, sem).wait()
            if not first:
                pltpu.make_async_copy(tb_hbm.at[sv], buf2, sem).wait()

        def scatter(buf, buf2, dv):
            cp = pltpu.async_copy(buf, acca.at[dv], sesc, add=True)
            pltpu.sync_copy(buf2, accb.at[dv], add=True)
            cp.wait()

        if first:
            @pl.loop(0, K)
            def _(r):
                ones = jnp.full((16,), 1.0, _f32)
                bufa2[r, pl.ds(0, 16)] = ones
                bufa2[r, pl.ds(16, 16)] = ones
                bufb2[r, pl.ds(0, 16)] = ones
                bufb2[r, pl.ds(16, 16)] = ones

        fetch(src_hbm, 0, s0, ss0)
        fetch(dst_hbm, 0, d0, sd0)
        fetch(src_hbm, 1, s1, ss1)
        fetch(dst_hbm, 1, d1, sd1)
        pltpu.sync_copy(za_hbm.at[pl.ds(sid * RZ, RZ)],
                        acca.at[pl.ds(sid * RZ, RZ)])
        pltpu.sync_copy(zb_hbm.at[pl.ds(sid * RZ, RZ)],
                        accb.at[pl.ds(sid * RZ, RZ)])
        wait_fetch(src_hbm, 0, s0, ss0)
        gather(s0, bufa, bufa2, sema)
        plsc.subcore_barrier()

        @pl.loop(0, CPW - 1, step=2)
        def _(j):
            wait_fetch(src_hbm, j + 1, s1, ss1)
            gather(s1, bufb, bufb2, semb)
            wait_gather(s0, bufa, bufa2, sema)

            @pl.when(j + 2 < CPW)
            def _():
                fetch(src_hbm, j + 2, s0, ss0)

            wait_fetch(dst_hbm, j, d0, sd0)
            scatter(bufa, bufa2, d0)

            @pl.when(j + 2 < CPW)
            def _():
                fetch(dst_hbm, j + 2, d0, sd0)
                wait_fetch(src_hbm, j + 2, s0, ss0)
                gather(s0, bufa, bufa2, sema)

            wait_gather(s1, bufb, bufb2, semb)

            @pl.when(j + 3 < CPW)
            def _():
                fetch(src_hbm, j + 3, s1, ss1)

            wait_fetch(dst_hbm, j + 1, d1, sd1)
            scatter(bufb, bufb2, d1)

            @pl.when(j + 3 < CPW)
            def _():
                fetch(dst_hbm, j + 3, d1, sd1)

        wait_gather(s0, bufa, bufa2, sema)
        wait_fetch(dst_hbm, CPW - 1, d0, sd0)
        scatter(bufa, bufa2, d0)

        plsc.subcore_barrier()
        pltpu.sync_copy(acca.at[pl.ds(sid * RZ, RZ)],
                        oa_hbm.at[pl.ds(cid * N_ACC + sid * RZ, RZ)])
        pltpu.sync_copy(accb.at[pl.ds(sid * RZ, RZ)],
                        ob_hbm.at[pl.ds(cid * N_ACC + sid * RZ, RZ),
                                  pl.ds(0, DB)])

    return agg


GRID = 4
R = N_ACC // GRID


def _dot_t(a, w):
    return lax.dot_general(a, w, (((1,), (1,)), ((), ())),
                           preferred_element_type=_f32)


def _spec(w, off=0):
    return pl.BlockSpec((R, w), lambda i, o=off: (i + o, 0))


def _wspec(r, c):
    return pl.BlockSpec((r, c), lambda i: (0, 0))


def _first_layer(aa, ab, x, wl, bl, wr):
    def body(a0, a1, c0, c1, x_ref, wl_ref, bl_ref, wr_ref,
             ha_ref, hb_ref, inv_ref):
        s = a0[...] + a1[...]
        inv = 1.0 / jnp.maximum(c0[:, 0:1] + c1[:, 0:1], 1.0)
        z = _dot_t(s * inv, wl_ref[...])
        z = z + bl_ref[0:1, :] + _dot_t(x_ref[...], wr_ref[...])
        z = jnp.maximum(z, 0.0)
        ha_ref[...] = z[:, :DA]
        hb_ref[...] = z[:, DA:]
        inv_ref[...] = jnp.broadcast_to(inv, (R, 16))

    return pl.pallas_call(
        body,
        grid=(GRID,),
        in_specs=[
            _spec(DA), _spec(DA, GRID), _spec(DA), _spec(DA, GRID),
            _spec(DA), _wspec(160, 128), _wspec(8, 160), _wspec(160, 128),
        ],
        out_specs=[_spec(DA), _spec(DB), _spec(16)],
        out_shape=[
            jax.ShapeDtypeStruct((N_ACC, DA), _f32),
            jax.ShapeDtypeStruct((N_ACC, DB), _f32),
            jax.ShapeDtypeStruct((N_ACC, 16), _f32),
        ],
    )(aa, aa, ab, ab, x, wl, bl, wr)


def _layer(aa, ab, inv, ha, hb, wl, bl, wr, relu):
    dout = wl.shape[0]

    def body(a0, a1, b0, b1, inv_ref, ha_ref, hb_ref,
             wl_ref, bl_ref, wr_ref, *outs):
        s = jnp.concatenate(
            [a0[...] + a1[...], b0[:, :DB] + b1[:, :DB]], axis=1)
        s = s * inv_ref[:, 0:1]
        h = jnp.concatenate([ha_ref[...], hb_ref[...]], axis=1)
        z = _dot_t(s, wl_ref[...])
        z = z + bl_ref[0:1, :] + _dot_t(h, wr_ref[...])
        if relu:
            z = jnp.maximum(z, 0.0)
        if len(outs) == 2:
            outs[0][...] = z[:, :DA]
            outs[1][...] = z[:, DA:]
        else:
            outs[0][...] = z

    if dout == 160:
        out_specs = [_spec(DA), _spec(DB)]
        out_shape = [jax.ShapeDtypeStruct((N_ACC, DA), _f32),
                     jax.ShapeDtypeStruct((N_ACC, DB), _f32)]
    else:
        out_specs = [_spec(dout)]
        out_shape = [jax.ShapeDtypeStruct((N_ACC, dout), _f32)]

    return pl.pallas_call(
        body,
        grid=(GRID,),
        in_specs=[
            _spec(DA), _spec(DA, GRID), _spec(DA), _spec(DA, GRID),
            _spec(16), _spec(DA), _spec(DB),
            _wspec(dout, 160), _wspec(8, dout), _wspec(dout, 160),
        ],
        out_specs=out_specs,
        out_shape=out_shape,
    )(aa, aa, ab, ab, inv, ha, hb, wl, bl, wr)


def _pool(h, batch2d):
    def body(h_ref, b_ref, out_ref):
        hv = h_ref[...]
        b = b_ref[0:1, :]
        gid = lax.broadcasted_iota(jnp.int32, (G, N_ACC), 0)
        onehot = (b == gid).astype(_f32)
        pooled = lax.dot_general(onehot, hv, (((1,), (0,)), ((), ())),
                                 preferred_element_type=_f32)
        gcnt = jnp.sum(onehot, axis=1, keepdims=True)
        out_ref[...] = pooled / jnp.maximum(gcnt, 1.0)

    return pl.pallas_call(
        body,
        grid=(1,),
        in_specs=[
            pl.BlockSpec((N_ACC, 128), lambda i: (0, 0)),
            pl.BlockSpec((8, N_ACC), lambda i: (0, 0)),
        ],
        out_specs=pl.BlockSpec((G, 128), lambda i: (0, 0)),
        out_shape=jax.ShapeDtypeStruct((G, 128), _f32),
    )(h, batch2d)


def _pad(a, r, c):
    return jnp.zeros((r, c), _f32).at[: a.shape[0], : a.shape[1]].set(a)


def kernel(x, edge_index, batch, W_l_in, b_l_in, W_r_in, W_l_mid, b_l_mid,
           W_r_mid, W_l_out, b_l_out, W_r_out):
    src = jnp.pad(edge_index[0].astype(jnp.int32).reshape(NW, EPW),
                  ((0, 0), (0, EPAD))).reshape(NW, CPW, K)
    dst = jnp.pad(edge_index[1].astype(jnp.int32).reshape(NW, EPW),
                  ((0, 0), (0, EPAD)),
                  constant_values=N).reshape(NW, CPW, K)

    xp = _pad(x, N_ACC, DA)
    za = jnp.zeros((N_ACC, DA), _f32)
    zb = jnp.zeros((N_ACC, DB), _f32)

    wl_in = _pad(W_l_in, 160, 128)
    wr_in = _pad(W_r_in, 160, 128)
    bl_in = jnp.broadcast_to(_pad(b_l_in[None, :], 1, 160), (8, 160))
    wl_mid = jnp.zeros((NMID, 160, 160), _f32).at[:, :150, :150].set(W_l_mid)
    wr_mid = jnp.zeros((NMID, 160, 160), _f32).at[:, :150, :150].set(W_r_mid)
    bl_mid = jnp.zeros((NMID, 8, 160), _f32).at[:, :, :150].set(
        jnp.broadcast_to(b_l_mid[:, None, :], (NMID, 8, 150)))
    wl_out = _pad(W_l_out, 128, 160)
    wr_out = _pad(W_r_out, 128, 160)
    bl_out = jnp.broadcast_to(_pad(b_l_out[None, :], 1, 128), (8, 128))

    batch2d = jnp.broadcast_to(
        jnp.pad(batch.astype(jnp.int32), (0, N_ACC - N),
                constant_values=G)[None, :], (8, N_ACC))

    agg_first = _make_agg(True)
    agg_mid = _make_agg(False)

    aa, ab = agg_first(xp, zb, src, dst, za, zb)
    ha, hb, inv = _first_layer(aa, ab, xp, wl_in, bl_in, wr_in)
    for i in range(NMID):
        aa, ab = agg_mid(ha, hb, src, dst, za, zb)
        ha, hb = _layer(aa, ab, inv, ha, hb,
                        wl_mid[i], bl_mid[i], wr_mid[i], True)
    aa, ab = agg_mid(ha, hb, src, dst, za, zb)
    h, = _layer(aa, ab, inv, ha, hb, wl_out, bl_out, wr_out, False)
    return _pool(h, batch2d)
